# Initial kernel scaffold; baseline (speedup 1.0000x reference)
import functools

import jax
import jax.numpy as jnp
from jax import lax
from jax.experimental import pallas as pl
from jax.experimental.pallas import tpu as pltpu
from jax.experimental.pallas import tpu_sc as plsc

N = 1048576
M = 1200
NC = 2
NS = 16
NW = NC * NS
L = 16
CHUNK = N // NW
SUB = 8192
NSUB = CHUNK // SUB
VPS = SUB // L

_mesh = plsc.VectorSubcoreMesh(
    core_axis_name="c", subcore_axis_name="s", num_cores=NC, num_subcores=NS)


@functools.partial(
    pl.kernel,
    out_type=jax.ShapeDtypeStruct((NW, 2 * M), jnp.float32),
    mesh=_mesh,
    scratch_types=[
        pltpu.VMEM((L * 2 * M,), jnp.float32),
        pltpu.VMEM((2 * M,), jnp.float32),
        pltpu.VMEM((SUB,), jnp.int32),
        pltpu.VMEM((SUB,), jnp.float32),
        pltpu.VMEM((SUB,), jnp.float32),
    ],
)
def _phase_a(x_hbm, y_hbm, m_hbm, part_hbm, bins, red, buf_m, buf_x, buf_y):
    wid = lax.axis_index("s") * NC + lax.axis_index("c")
    base = wid * CHUNK

    neg = jnp.full((L,), -jnp.inf, jnp.float32)
    lane = lax.broadcasted_iota(jnp.int32, (L,), 0)
    lane_base = lane * (2 * M)

    def init_body(j, _):
        bins[pl.ds(j * L, L)] = neg
        return 0
    lax.fori_loop(0, (L * 2 * M) // L, init_body, 0)

    def scan_body(t, carry):
        cur_m, cmx, cmy = carry
        off = t * L
        m = buf_m[pl.ds(off, L)]
        x = buf_x[pl.ds(off, L)]
        y = buf_y[pl.ds(off, L)]
        changed = m != cur_m
        idx = lane_base + cur_m
        plsc.store_scatter(bins, [idx], cmx, mask=changed)
        plsc.store_scatter(bins, [idx + M], cmy, mask=changed)
        cmx = jnp.where(changed, x, jnp.maximum(cmx, x))
        cmy = jnp.where(changed, y, jnp.maximum(cmy, y))
        return m, cmx, cmy

    carry = None
    for s in range(NSUB):
        off = base + s * SUB
        pltpu.sync_copy(m_hbm.at[pl.ds(off, SUB)], buf_m)
        pltpu.sync_copy(x_hbm.at[pl.ds(off, SUB)], buf_x)
        pltpu.sync_copy(y_hbm.at[pl.ds(off, SUB)], buf_y)
        if s == 0:
            carry = (buf_m[pl.ds(0, L)], buf_x[pl.ds(0, L)], buf_y[pl.ds(0, L)])
            t0 = 1
        else:
            t0 = 0
        carry = lax.fori_loop(t0, VPS, scan_body, carry)

    cur_m, cmx, cmy = carry
    idx = lane_base + cur_m
    plsc.store_scatter(bins, [idx], cmx)
    plsc.store_scatter(bins, [idx + M], cmy)

    def red_body(g, _):
        acc = neg
        for ln in range(L):
            acc = jnp.maximum(acc, bins[pl.ds(ln * (2 * M) + g * L, L)])
        red[pl.ds(g * L, L)] = acc
        return 0
    lax.fori_loop(0, (2 * M) // L, red_body, 0)

    pltpu.sync_copy(red, part_hbm.at[wid])


def _phase_b_body(p_ref, o_ref):
    p = p_ref[...]
    mx = jnp.max(p[:, :M], axis=0)
    my = jnp.max(p[:, M:], axis=0)
    d = mx - my
    o_ref[0, 0] = jnp.sum(d * d) * (1.0 / M)


_phase_b = pl.pallas_call(
    _phase_b_body, out_shape=jax.ShapeDtypeStruct((1, 1), jnp.float32))


def kernel(output, target, months):
    x = output.reshape(-1)
    y = target.reshape(-1)
    part = _phase_a(x, y, months)
    loss = _phase_b(part)
    return loss[0, 0]


# speedup vs baseline: 25.6361x; 25.6361x over previous
import functools

import jax
import jax.numpy as jnp
from jax import lax
from jax.experimental import pallas as pl
from jax.experimental.pallas import tpu as pltpu
from jax.experimental.pallas import tpu_sc as plsc

N = 1048576
M = 1200
NC = 2
NS = 16
NW = NC * NS
L = 16
CHUNK = N // NW
SUB = 8192
NSUB = CHUNK // SUB
VPS = SUB // L
LS = 2 * M + 16

_mesh = plsc.VectorSubcoreMesh(
    core_axis_name="c", subcore_axis_name="s", num_cores=NC, num_subcores=NS)


@functools.partial(
    pl.kernel,
    out_type=jax.ShapeDtypeStruct((NW, 2 * M), jnp.float32),
    mesh=_mesh,
    compiler_params=pltpu.CompilerParams(needs_layout_passes=False),
    scratch_types=[
        pltpu.VMEM((L * LS,), jnp.float32),
        pltpu.VMEM((2 * M,), jnp.float32),
        pltpu.VMEM((SUB,), jnp.int32),
        pltpu.VMEM((SUB,), jnp.float32),
        pltpu.VMEM((SUB,), jnp.float32),
    ],
)
def _phase_a(x_hbm, y_hbm, m_hbm, part_hbm, bins, red, buf_m, buf_x, buf_y):
    wid = lax.axis_index("s") * NC + lax.axis_index("c")
    base = wid * CHUNK

    neg = jnp.full((L,), -jnp.inf, jnp.float32)
    lane = lax.broadcasted_iota(jnp.int32, (L,), 0)
    lane_base = lane * LS

    def init_body(j, _):
        bins[pl.ds(j * L, L)] = neg
        return 0
    lax.fori_loop(0, (L * LS) // L, init_body, 0)

    def scan_body(t, carry):
        cur_m, cmx, cmy = carry
        off = t * L
        m = buf_m[pl.ds(off, L)]
        x = buf_x[pl.ds(off, L)]
        y = buf_y[pl.ds(off, L)]
        changed = m != cur_m
        ix = jnp.where(changed, lane_base + cur_m, lane_base + 2 * M)
        iy = jnp.where(changed, lane_base + M + cur_m, lane_base + 2 * M + 1)
        plsc.store_scatter(bins, [ix], cmx)
        plsc.store_scatter(bins, [iy], cmy)
        cmx = jnp.where(changed, x, jnp.maximum(cmx, x))
        cmy = jnp.where(changed, y, jnp.maximum(cmy, y))
        return m, cmx, cmy

    carry = None
    for s in range(NSUB):
        off = base + s * SUB
        pltpu.sync_copy(m_hbm.at[pl.ds(off, SUB)], buf_m)
        pltpu.sync_copy(x_hbm.at[pl.ds(off, SUB)], buf_x)
        pltpu.sync_copy(y_hbm.at[pl.ds(off, SUB)], buf_y)
        if s == 0:
            carry = (buf_m[pl.ds(0, L)], buf_x[pl.ds(0, L)], buf_y[pl.ds(0, L)])
            t0 = 1
        else:
            t0 = 0
        carry = lax.fori_loop(t0, VPS, scan_body, carry)

    cur_m, cmx, cmy = carry
    idx = lane_base + cur_m
    plsc.store_scatter(bins, [idx], cmx)
    plsc.store_scatter(bins, [idx + M], cmy)

    def red_body(g, _):
        acc = neg
        for ln in range(L):
            acc = jnp.maximum(acc, bins[pl.ds(ln * LS + g * L, L)])
        red[pl.ds(g * L, L)] = acc
        return 0
    lax.fori_loop(0, (2 * M) // L, red_body, 0)

    pltpu.sync_copy(red, part_hbm.at[wid])


def _phase_b_body(p_ref, o_ref):
    p = p_ref[...]
    mx = jnp.max(p[:, :M], axis=0)
    my = jnp.max(p[:, M:], axis=0)
    d = mx - my
    o_ref[...] = (jnp.sum(d * d) * (1.0 / M))[None, None]


_phase_b = pl.pallas_call(
    _phase_b_body, out_shape=jax.ShapeDtypeStruct((1, 1), jnp.float32))


def kernel(output, target, months):
    x = output.reshape(-1)
    y = target.reshape(-1)
    part = _phase_a(x, y, months)
    loss = _phase_b(part)
    return loss[0, 0]


# speedup vs baseline: 36.0926x; 1.4079x over previous
import functools

import jax
import jax.numpy as jnp
from jax import lax
from jax.experimental import pallas as pl
from jax.experimental.pallas import tpu as pltpu
from jax.experimental.pallas import tpu_sc as plsc

N = 1048576
M = 1200
NC = 2
NS = 16
NW = NC * NS
L = 16
CHUNK = N // NW
SUB = 8192
NSUB = CHUNK // SUB
VPS = SUB // L
U = 8
LS = 2 * M

_mesh = plsc.VectorSubcoreMesh(
    core_axis_name="c", subcore_axis_name="s", num_cores=NC, num_subcores=NS)


@functools.partial(
    pl.kernel,
    out_type=jax.ShapeDtypeStruct((NW, 2 * M), jnp.float32),
    mesh=_mesh,
    compiler_params=pltpu.CompilerParams(needs_layout_passes=False),
    scratch_types=[
        pltpu.VMEM((L * LS,), jnp.float32),
        pltpu.VMEM((2 * M,), jnp.float32),
        pltpu.VMEM((2, SUB), jnp.int32),
        pltpu.VMEM((2, SUB), jnp.float32),
        pltpu.VMEM((2, SUB), jnp.float32),
        pltpu.SemaphoreType.DMA,
        pltpu.SemaphoreType.DMA,
    ],
)
def _phase_a(x_hbm, y_hbm, m_hbm, part_hbm, bins, red, buf_m, buf_x, buf_y,
             sem0, sem1):
    wid = lax.axis_index("s") * NC + lax.axis_index("c")
    base = wid * CHUNK
    sems = (sem0, sem1)

    neg = jnp.full((L,), -jnp.inf, jnp.float32)
    lane = lax.broadcasted_iota(jnp.int32, (L,), 0)
    lane_base = lane * LS

    def init_body(j, _):
        for u in range(U):
            bins[pl.ds(j * (U * L) + u * L, L)] = neg
        return 0
    lax.fori_loop(0, (L * LS) // (U * L), init_body, 0)

    def copies(s):
        par = s % 2
        off = base + s * SUB
        sem = sems[par]
        return [
            pltpu.make_async_copy(m_hbm.at[pl.ds(off, SUB)], buf_m.at[par], sem),
            pltpu.make_async_copy(x_hbm.at[pl.ds(off, SUB)], buf_x.at[par], sem),
            pltpu.make_async_copy(y_hbm.at[pl.ds(off, SUB)], buf_y.at[par], sem),
        ]

    def scan_step(par, voff, carry):
        cur_m, cmx, cmy = carry
        m = buf_m[par, pl.ds(voff, L)]
        x = buf_x[par, pl.ds(voff, L)]
        y = buf_y[par, pl.ds(voff, L)]
        changed = m != cur_m
        ix = lane_base + cur_m
        plsc.store_scatter(bins, [ix], cmx, mask=changed)
        plsc.store_scatter(bins, [ix + M], cmy, mask=changed)
        cmx = jnp.where(changed, x, jnp.maximum(cmx, x))
        cmy = jnp.where(changed, y, jnp.maximum(cmy, y))
        return m, cmx, cmy

    for c in copies(0):
        c.start()

    carry = None
    for s in range(NSUB):
        par = s % 2
        if s + 1 < NSUB:
            nxt = copies(s + 1)
            for c in nxt:
                c.start()
        for c in copies(s):
            c.wait()
        if s == 0:
            carry = (buf_m[0, pl.ds(0, L)], buf_x[0, pl.ds(0, L)],
                     buf_y[0, pl.ds(0, L)])
            for t in range(1, U):
                carry = scan_step(0, t * L, carry)
            t0 = 1
        else:
            t0 = 0

        def block_body(i, carry, par=par):
            for u in range(U):
                carry = scan_step(par, i * (U * L) + u * L, carry)
            return carry
        carry = lax.fori_loop(t0, VPS // U, block_body, carry)

    cur_m, cmx, cmy = carry
    ix = lane_base + cur_m
    plsc.store_scatter(bins, [ix], cmx)
    plsc.store_scatter(bins, [ix + M], cmy)

    def red_body(g, _):
        acc = neg
        for ln in range(L):
            acc = jnp.maximum(acc, bins[pl.ds(ln * LS + g * L, L)])
        red[pl.ds(g * L, L)] = acc
        return 0
    lax.fori_loop(0, (2 * M) // L, red_body, 0)

    pltpu.sync_copy(red, part_hbm.at[wid])


def _phase_b_body(p_ref, o_ref):
    p = p_ref[...]
    mx = jnp.max(p[:, :M], axis=0)
    my = jnp.max(p[:, M:], axis=0)
    d = mx - my
    o_ref[...] = (jnp.sum(d * d) * (1.0 / M))[None, None]


_phase_b = pl.pallas_call(
    _phase_b_body, out_shape=jax.ShapeDtypeStruct((1, 1), jnp.float32))


def kernel(output, target, months):
    x = output.reshape(-1)
    y = target.reshape(-1)
    part = _phase_a(x, y, months)
    loss = _phase_b(part)
    return loss[0, 0]
